# baseline (device time: 363645 ns/iter reference)
import jax
import jax.numpy as jnp
from jax import lax
from jax.experimental import pallas as pl
from jax.experimental.pallas import tpu as pltpu

N_DEV = 16
M_CHUNK = 512
N_SUB = 4


def kernel(x, w_mat):
    m, k_shard = x.shape
    _, n = w_mat.shape
    hn = n // 2
    sw = hn // N_SUB

    def body(x_ref, w_ref, out_ref, comm_r, comm_l,
             send_r, recv_r, send_l, recv_l):
        d = lax.axis_index("i")
        left = lax.rem(d - 1 + N_DEV, N_DEV)
        right = lax.rem(d + 1, N_DEV)

        def sub0_fill_parts():
            p0_r = jnp.dot(
                x_ref[pl.ds(lax.rem(d - 1 + N_DEV, N_DEV) * M_CHUNK, M_CHUNK), :],
                w_ref[:, : n // (2 * N_SUB)],
                preferred_element_type=jnp.float32)
            p0_l = jnp.dot(
                x_ref[pl.ds(lax.rem(d + 1, N_DEV) * M_CHUNK, M_CHUNK), :],
                w_ref[:, n // 2: n // 2 + n // (2 * N_SUB)],
                preferred_element_type=jnp.float32)
            return p0_r, p0_l

        p0 = dict(zip(("r", "l"), sub0_fill_parts()))

        barrier_sem = pltpu.get_barrier_semaphore()
        for nbr in (left, right):
            pl.semaphore_signal(
                barrier_sem, inc=1,
                device_id=(nbr,), device_id_type=pl.DeviceIdType.MESH,
            )
        pl.semaphore_wait(barrier_sem, 2)

        def parts_for_step(s):
            c_r = lax.rem(d - 1 - s + 2 * N_DEV, N_DEV)
            c_l = lax.rem(d + 1 + s, N_DEV)
            p_r = jnp.dot(x_ref[pl.ds(c_r * M_CHUNK, M_CHUNK), :],
                          w_ref[:, :hn], preferred_element_type=jnp.float32)
            p_l = jnp.dot(x_ref[pl.ds(c_l * M_CHUNK, M_CHUNK), :],
                          w_ref[:, hn:], preferred_element_type=jnp.float32)
            return p_r, p_l

        def final_parts():
            xs = x_ref[pl.ds(d * M_CHUNK, M_CHUNK), :]
            p_r = jnp.dot(xs, w_ref[:, :hn], preferred_element_type=jnp.float32)
            p_l = jnp.dot(xs, w_ref[:, hn:], preferred_element_type=jnp.float32)
            return p_r, p_l

        def make_rdma(comm, send, recv, ss, sub, target):
            return pltpu.make_async_remote_copy(
                src_ref=comm.at[ss, :, pl.ds(sub * sw, sw)],
                dst_ref=comm.at[(ss + 1) % 2, :, pl.ds(sub * sw, sw)],
                send_sem=send.at[ss, sub], recv_sem=recv.at[(ss + 1) % 2, sub],
                device_id=(target,), device_id_type=pl.DeviceIdType.MESH,
            )

        part_r = part_l = None
        prev = {}
        for s in range(N_DEV - 1):
            ss = s % 2
            for sub in range(N_SUB):
                cols = pl.ds(sub * sw, sw)
                for key, comm, send, recv, part, tgt in (
                    ("r", comm_r, send_r, recv_r, part_r, right),
                    ("l", comm_l, send_l, recv_l, part_l, left),
                ):
                    if s == 0:
                        if sub == 0:
                            p = p0[key]
                        else:
                            c0 = (d - 1 if key == "r" else d + 1) + 2 * N_DEV
                            off = (0 if key == "r" else hn) + sub * sw
                            p = jnp.dot(
                                x_ref[pl.ds(lax.rem(c0, N_DEV) * M_CHUNK,
                                            M_CHUNK), :],
                                w_ref[:, off:off + sw],
                                preferred_element_type=jnp.float32)
                        comm[ss, :, cols] = p.astype(jnp.bfloat16)
                    else:
                        prev[key, sub].wait()
                        comm[ss, :, cols] = (
                            part[:, sub * sw:(sub + 1) * sw]
                            + comm[ss, :, cols].astype(jnp.float32)
                        ).astype(jnp.bfloat16)
                    rdma = make_rdma(comm, send, recv, ss, sub, tgt)
                    rdma.start()
                    prev[key, sub] = rdma
            if s < N_DEV - 2:
                part_r, part_l = parts_for_step(s + 1)
            else:
                part_r, part_l = final_parts()

        last = (N_DEV - 1) % 2
        for sub in range(N_SUB):
            cols = pl.ds(sub * sw, sw)
            prev["r", sub].wait()
            out_ref[:, sub * sw:(sub + 1) * sw] = jnp.maximum(
                part_r[:, sub * sw:(sub + 1) * sw]
                + comm_r[last, :, cols].astype(jnp.float32), 0.0)
            prev["l", sub].wait()
            out_ref[:, hn + sub * sw:hn + (sub + 1) * sw] = jnp.maximum(
                part_l[:, sub * sw:(sub + 1) * sw]
                + comm_l[last, :, cols].astype(jnp.float32), 0.0)

    return pl.pallas_call(
        body,
        out_shape=jax.ShapeDtypeStruct((M_CHUNK, n), jnp.float32),
        in_specs=[
            pl.BlockSpec(memory_space=pltpu.VMEM),
            pl.BlockSpec(memory_space=pltpu.VMEM),
        ],
        out_specs=pl.BlockSpec(memory_space=pltpu.VMEM),
        scratch_shapes=[
            pltpu.VMEM((2, M_CHUNK, hn), jnp.bfloat16),
            pltpu.VMEM((2, M_CHUNK, hn), jnp.bfloat16),
            pltpu.SemaphoreType.DMA((2, N_SUB)),
            pltpu.SemaphoreType.DMA((2, N_SUB)),
            pltpu.SemaphoreType.DMA((2, N_SUB)),
            pltpu.SemaphoreType.DMA((2, N_SUB)),
        ],
        compiler_params=pltpu.CompilerParams(
            collective_id=0, vmem_limit_bytes=96 * 1024 * 1024
        ),
    )(x, w_mat)


# device time: 363406 ns/iter; 1.0007x vs baseline; 1.0007x over previous
import jax
import jax.numpy as jnp
from jax import lax
from jax.experimental import pallas as pl
from jax.experimental.pallas import tpu as pltpu

N_DEV = 16
M_CHUNK = 512
N_SUB = 4


def kernel(x, w_mat):
    m, k_shard = x.shape
    _, n = w_mat.shape
    hn = n // 2
    sw = hn // N_SUB

    def body(x_ref, w_ref, out_ref, comm_r, comm_l,
             send_r, recv_r, send_l, recv_l):
        d = lax.axis_index("i")
        left = lax.rem(d - 1 + N_DEV, N_DEV)
        right = lax.rem(d + 1, N_DEV)

        barrier_sem = pltpu.get_barrier_semaphore()
        for nbr in (left, right):
            pl.semaphore_signal(
                barrier_sem, inc=1,
                device_id=(nbr,), device_id_type=pl.DeviceIdType.MESH,
            )
        pl.semaphore_wait(barrier_sem, 2)

        def parts_for_step(s):
            c_r = lax.rem(d - 1 - s + 2 * N_DEV, N_DEV)
            c_l = lax.rem(d + 1 + s, N_DEV)
            p_r = jnp.dot(x_ref[pl.ds(c_r * M_CHUNK, M_CHUNK), :],
                          w_ref[:, :hn], preferred_element_type=jnp.float32)
            p_l = jnp.dot(x_ref[pl.ds(c_l * M_CHUNK, M_CHUNK), :],
                          w_ref[:, hn:], preferred_element_type=jnp.float32)
            return p_r, p_l

        def final_parts():
            xs = x_ref[pl.ds(d * M_CHUNK, M_CHUNK), :]
            p_r = jnp.dot(xs, w_ref[:, :hn], preferred_element_type=jnp.float32)
            p_l = jnp.dot(xs, w_ref[:, hn:], preferred_element_type=jnp.float32)
            return p_r, p_l

        def make_rdma(comm, send, recv, ss, sub, target):
            return pltpu.make_async_remote_copy(
                src_ref=comm.at[ss, :, pl.ds(sub * sw, sw)],
                dst_ref=comm.at[(ss + 1) % 2, :, pl.ds(sub * sw, sw)],
                send_sem=send.at[ss, sub], recv_sem=recv.at[(ss + 1) % 2, sub],
                device_id=(target,), device_id_type=pl.DeviceIdType.MESH,
            )

        part_r = part_l = None
        prev = {}
        for s in range(N_DEV - 1):
            ss = s % 2
            for sub in range(N_SUB):
                cols = pl.ds(sub * sw, sw)
                for key, comm, send, recv, part, tgt in (
                    ("r", comm_r, send_r, recv_r, part_r, right),
                    ("l", comm_l, send_l, recv_l, part_l, left),
                ):
                    if s == 0:
                        c0 = (d - 1 if key == "r" else d + 1) + 2 * N_DEV
                        off = (0 if key == "r" else hn) + sub * sw
                        p = jnp.dot(
                            x_ref[pl.ds(lax.rem(c0, N_DEV) * M_CHUNK, M_CHUNK), :],
                            w_ref[:, off:off + sw],
                            preferred_element_type=jnp.float32)
                        comm[ss, :, cols] = p.astype(jnp.bfloat16)
                    else:
                        prev[key, sub].wait()
                        comm[ss, :, cols] = (
                            part[:, sub * sw:(sub + 1) * sw]
                            + comm[ss, :, cols].astype(jnp.float32)
                        ).astype(jnp.bfloat16)
                    rdma = make_rdma(comm, send, recv, ss, sub, tgt)
                    rdma.start()
                    prev[key, sub] = rdma
            if s < N_DEV - 2:
                part_r, part_l = parts_for_step(s + 1)
            else:
                part_r, part_l = final_parts()

        last = (N_DEV - 1) % 2
        for sub in range(N_SUB):
            cols = pl.ds(sub * sw, sw)
            prev["r", sub].wait()
            out_ref[:, sub * sw:(sub + 1) * sw] = jnp.maximum(
                part_r[:, sub * sw:(sub + 1) * sw]
                + comm_r[last, :, cols].astype(jnp.float32), 0.0)
            prev["l", sub].wait()
            out_ref[:, hn + sub * sw:hn + (sub + 1) * sw] = jnp.maximum(
                part_l[:, sub * sw:(sub + 1) * sw]
                + comm_l[last, :, cols].astype(jnp.float32), 0.0)

    return pl.pallas_call(
        body,
        out_shape=jax.ShapeDtypeStruct((M_CHUNK, n), jnp.float32),
        in_specs=[
            pl.BlockSpec(memory_space=pltpu.VMEM),
            pl.BlockSpec(memory_space=pltpu.VMEM),
        ],
        out_specs=pl.BlockSpec(memory_space=pltpu.VMEM),
        scratch_shapes=[
            pltpu.VMEM((2, M_CHUNK, hn), jnp.bfloat16),
            pltpu.VMEM((2, M_CHUNK, hn), jnp.bfloat16),
            pltpu.SemaphoreType.DMA((2, N_SUB)),
            pltpu.SemaphoreType.DMA((2, N_SUB)),
            pltpu.SemaphoreType.DMA((2, N_SUB)),
            pltpu.SemaphoreType.DMA((2, N_SUB)),
        ],
        compiler_params=pltpu.CompilerParams(
            collective_id=0, vmem_limit_bytes=96 * 1024 * 1024
        ),
    )(x, w_mat)
